# baseline (speedup 1.0000x reference)
import functools

import jax
import jax.numpy as jnp
from jax import lax
from jax.experimental import pallas as pl
from jax.experimental.pallas import tpu as pltpu
from jax.experimental.pallas import tpu_sc as plsc

DIM = 64
GAMMA = 12.0
L = 16
NCHUNK = DIM // L
NC, NS = 2, 16
NW = NC * NS
STG = 133


@functools.lru_cache(maxsize=None)
def _make_tc_transpose(NENT):
    BLK = 512
    nblk = -(-(NENT // 2) // BLK)
    F = nblk * BLK

    def body(x0_ref, x1_ref, o_ref):
        eye = jnp.eye(DIM, dtype=jnp.float32)
        dn = (((0,), (0,)), ((), ()))
        t0 = lax.dot_general(
            x0_ref[...], eye, dn, precision=lax.Precision.HIGHEST)
        t1 = lax.dot_general(
            x1_ref[...], eye, dn, precision=lax.Precision.HIGHEST)
        o_ref[...] = jnp.concatenate([t0, t1], axis=1)

    call = pl.pallas_call(
        body,
        out_shape=jax.ShapeDtypeStruct((F, 2 * DIM), jnp.float32),
        grid=(nblk,),
        in_specs=[
            pl.BlockSpec((DIM, BLK), lambda i: (0, i)),
            pl.BlockSpec((DIM, BLK), lambda i: (0, i + nblk)),
        ],
        out_specs=pl.BlockSpec((BLK, 2 * DIM), lambda i: (i, 0)),
    )
    return call, F


@functools.lru_cache(maxsize=None)
def _make_score_kernel(B, NEG, F):
    rows_pw = B // NW
    TPT = 128
    halves = NEG // TPT
    tasks_pw = rows_pw * halves

    mesh = plsc.VectorSubcoreMesh(
        core_axis_name="c", subcore_axis_name="s",
        num_cores=NC, num_subcores=NS)

    @functools.partial(
        pl.kernel,
        out_type=jax.ShapeDtypeStruct((B * halves, TPT), jnp.float32),
        mesh=mesh,
        compiler_params=pltpu.CompilerParams(
            needs_layout_passes=False, use_tc_tiling_on_sc=True),
        scratch_types=[
            pltpu.VMEM((rows_pw,), jnp.int32),
            pltpu.VMEM((rows_pw,), jnp.int32),
            pltpu.VMEM((rows_pw,), jnp.int32),
            pltpu.VMEM((rows_pw,), jnp.int32),
            pltpu.VMEM((rows_pw, 2 * DIM), jnp.float32),
            pltpu.VMEM((rows_pw, 2 * DIM), jnp.float32),
            pltpu.VMEM((rows_pw, DIM), jnp.float32),
            pltpu.VMEM((tasks_pw, TPT), jnp.int32),
            pltpu.VMEM((TPT,), jnp.int32),
            pltpu.VMEM((TPT,), jnp.int32),
            pltpu.VMEM((TPT, 2 * DIM), jnp.float32),
            pltpu.VMEM((TPT, 2 * DIM), jnp.float32),
            pltpu.VMEM((TPT,), jnp.float32),
            pltpu.VMEM((TPT,), jnp.float32),
            pltpu.SemaphoreType.DMA,
            pltpu.SemaphoreType.DMA,
            pltpu.SemaphoreType.DMA,
            pltpu.SemaphoreType.DMA,
            pltpu.SemaphoreType.DMA,
        ],
    )
    def k(hidx_hbm, ridx_hbm, tidx_hbm, ent_hbm, rel_hbm, out_hbm,
          hidx_v, ridx_v, hg_v, rg_v, head_v, relv_v, hr_v, tidx_v,
          gidx0, gidx1, tails0, tails1, scores0, scores1,
          gsem0, gsem1, osem0, osem1, psem):
        wid = lax.axis_index("s") * NC + lax.axis_index("c")
        base_row = wid * rows_pw
        base_task = wid * tasks_pw
        lane_iota = lax.iota(jnp.int32, L)

        pltpu.sync_copy(hidx_hbm.at[pl.ds(base_row, rows_pw)], hidx_v)
        pltpu.sync_copy(ridx_hbm.at[pl.ds(base_row, rows_pw)], ridx_v)
        for c in range(rows_pw // L):
            sl = pl.ds(c * L, L)
            hch = hidx_v[sl]
            hg_v[sl] = jnp.where(hch >= F, hch - F, hch)
            rg_v[sl] = ridx_v[sl] >> 1
        cp_t = pltpu.async_copy(
            tidx_hbm.at[pl.ds(base_task, tasks_pw)], tidx_v, psem)
        cp_h = pltpu.async_copy(ent_hbm.at[hg_v], head_v, psem)
        cp_r = pltpu.async_copy(rel_hbm.at[rg_v], relv_v, psem)
        cp_t.wait()
        cp_h.wait()
        cp_r.wait()

        for rg in range(rows_pw // L):
            rows = rg * L + lane_iota
            hoffs = jnp.where(hidx_v[pl.ds(rg * L, L)] >= F, DIM, 0)
            roffs = (ridx_v[pl.ds(rg * L, L)] & 1) * DIM
            for d in range(DIM):
                hv = plsc.load_gather(head_v, [rows, hoffs + d])
                rv = plsc.load_gather(relv_v, [rows, roffs + d])
                plsc.store_scatter(
                    hr_v, [rows, jnp.full((L,), d, jnp.int32)], hv + rv)

        def fill_gidx(t, gidx):
            for c in range(TPT // L):
                sl = pl.ds(c * L, L)
                tch = tidx_v[t, sl]
                gidx[sl] = jnp.where(tch >= F, tch - F, tch)

        fill_gidx(0, gidx0)
        fill_gidx(1, gidx1)
        pltpu.async_copy(ent_hbm.at[gidx0], tails0, gsem0)
        pltpu.async_copy(ent_hbm.at[gidx1], tails1, gsem1)

        def run_task(i, par, gidx, tails, scores, gsem, osem):
            t = halves * i + par
            pltpu.make_async_copy(ent_hbm.at[gidx], tails, gsem).wait()
            @pl.when(i > 0)
            def _():
                pltpu.make_async_copy(
                    scores, out_hbm.at[base_task], osem).wait()

            rowvec_i = jnp.full((L,), i, jnp.int32)

            def group_body(g, carry):
                sbase = g * L
                rows = sbase + lane_iota
                cols0 = jnp.where(tidx_v[t, pl.ds(sbase, L)] >= F, DIM, 0)
                acc0 = jnp.zeros((L,), jnp.float32)
                acc1 = jnp.zeros((L,), jnp.float32)
                for step in range(DIM):
                    dvec = (lane_iota + step) & (DIM - 1)
                    vals = plsc.load_gather(tails, [rows, cols0 + dvec])
                    hrv = plsc.load_gather(hr_v, [rowvec_i, dvec])
                    if step % 2 == 0:
                        acc0 = acc0 + jnp.abs(hrv - vals)
                    else:
                        acc1 = acc1 + jnp.abs(hrv - vals)
                scores[pl.ds(sbase, L)] = GAMMA - (acc0 + acc1)
                return carry
            lax.fori_loop(0, TPT // L, group_body, 0)

            pltpu.async_copy(scores, out_hbm.at[base_task + t], osem)
            @pl.when(i < rows_pw - 1)
            def _():
                fill_gidx(t + halves, gidx)
                pltpu.async_copy(ent_hbm.at[gidx], tails, gsem)

        def loop_body(i, carry):
            run_task(i, 0, gidx0, tails0, scores0, gsem0, osem0)
            run_task(i, 1, gidx1, tails1, scores1, gsem1, osem1)
            return carry
        lax.fori_loop(0, rows_pw, loop_body, 0)

        pltpu.make_async_copy(scores0, out_hbm.at[base_task], osem0).wait()
        pltpu.make_async_copy(scores1, out_hbm.at[base_task], osem1).wait()

    return k


def kernel(head_part, tail_part, relative_dist, entity_embedding,
           relation_embedding, relation_head, relation_tail):
    B, NEG = tail_part.shape
    NENT, D = entity_embedding.shape
    NRELT = relation_embedding.shape[0]

    tc_call, F = _make_tc_transpose(NENT)
    ent_t = entity_embedding.T
    ent2 = tc_call(ent_t, ent_t)

    h_idx = head_part[:, 0].astype(jnp.int32)
    r_idx = head_part[:, 1].astype(jnp.int32)
    tidx = tail_part.astype(jnp.int32).reshape(B * (NEG // 128), 128)
    rel2 = relation_embedding.reshape(NRELT // 2, 2 * D)
    k = _make_score_kernel(B, NEG, F)
    out = k(h_idx, r_idx, tidx, ent2, rel2)
    return out.reshape(B, NEG)


# speedup vs baseline: 1.4980x; 1.4980x over previous
import functools

import jax
import jax.numpy as jnp
from jax import lax
from jax.experimental import pallas as pl
from jax.experimental.pallas import tpu as pltpu
from jax.experimental.pallas import tpu_sc as plsc

DIM = 64
GAMMA = 12.0
L = 16
NCHUNK = DIM // L


def _lane_sum(acc):
    return jnp.sum(acc)


@functools.lru_cache(maxsize=None)
def _make_sc_kernel(B, NEG, interpret=False):
    NC, NS = 2, 16
    NW = NC * NS
    rows_pw = B // NW
    TPT = 128
    halves = NEG // TPT
    tasks_pw = rows_pw * halves

    mesh = plsc.VectorSubcoreMesh(
        core_axis_name="c", subcore_axis_name="s",
        num_cores=NC, num_subcores=NS)

    @functools.partial(
        pl.kernel,
        out_type=jax.ShapeDtypeStruct((B * halves, TPT), jnp.float32),
        mesh=mesh,
        interpret=interpret,
        compiler_params=pltpu.CompilerParams(
            needs_layout_passes=False, use_tc_tiling_on_sc=False),
        scratch_types=[
            pltpu.VMEM((rows_pw,), jnp.int32),
            pltpu.VMEM((rows_pw,), jnp.int32),
            pltpu.VMEM((rows_pw, DIM), jnp.float32),
            pltpu.VMEM((rows_pw, DIM), jnp.float32),
            pltpu.VMEM((tasks_pw, TPT), jnp.int32),
            pltpu.VMEM((TPT, DIM), jnp.float32),
            pltpu.VMEM((TPT, DIM), jnp.float32),
            pltpu.VMEM((TPT,), jnp.float32),
            pltpu.VMEM((TPT,), jnp.float32),
            pltpu.SemaphoreType.DMA,
            pltpu.SemaphoreType.DMA,
            pltpu.SemaphoreType.DMA,
            pltpu.SemaphoreType.DMA,
            pltpu.SemaphoreType.DMA,
        ],
    )
    def k(hidx_hbm, ridx_hbm, tidx_hbm, ent_hbm, rel_hbm, out_hbm,
          hidx_v, ridx_v, head_v, relv_v, tidx_v, tails0, tails1,
          scores0, scores1, gsem0, gsem1, osem0, osem1, psem):
        wid = lax.axis_index("s") * NC + lax.axis_index("c")
        base_row = wid * rows_pw
        base_task = wid * tasks_pw

        pltpu.sync_copy(hidx_hbm.at[pl.ds(base_row, rows_pw)], hidx_v)
        pltpu.sync_copy(ridx_hbm.at[pl.ds(base_row, rows_pw)], ridx_v)
        cp_t = pltpu.async_copy(
            tidx_hbm.at[pl.ds(base_task, tasks_pw)], tidx_v, psem)
        cp_h = pltpu.async_copy(ent_hbm.at[hidx_v], head_v, psem)
        cp_r = pltpu.async_copy(rel_hbm.at[ridx_v], relv_v, psem)
        cp_t.wait()
        cp_h.wait()
        cp_r.wait()

        def add_body(i, carry):
            for c in range(NCHUNK):
                sl = pl.ds(c * L, L)
                head_v[i, sl] = head_v[i, sl] + relv_v[i, sl]
            return carry
        lax.fori_loop(0, rows_pw, add_body, 0)

        pltpu.async_copy(ent_hbm.at[tidx_v.at[0]], tails0, gsem0)
        pltpu.async_copy(ent_hbm.at[tidx_v.at[1]], tails1, gsem1)

        lane_iota = lax.iota(jnp.int32, L)

        def run_task(i, par, tails, scores, gsem, osem):
            t = halves * i + par
            pltpu.make_async_copy(ent_hbm.at[tidx_v.at[t]], tails, gsem).wait()
            @pl.when(i > 0)
            def _():
                pltpu.make_async_copy(
                    scores, out_hbm.at[base_task], osem).wait()

            hr = [head_v[i, pl.ds(c * L, L)] for c in range(NCHUNK)]

            def group_body(g, carry):
                sbase = g * L
                svec = jnp.zeros((L,), jnp.float32)
                for j in range(L):
                    tt = sbase + j
                    acc = jnp.abs(hr[0] - tails[tt, pl.ds(0, L)])
                    for c in range(1, NCHUNK):
                        acc = acc + jnp.abs(hr[c] - tails[tt, pl.ds(c * L, L)])
                    s = GAMMA - _lane_sum(acc)
                    svec = jnp.where(lane_iota == j, s, svec)
                scores[pl.ds(sbase, L)] = svec
                return carry
            lax.fori_loop(0, TPT // L, group_body, 0)

            pltpu.async_copy(scores, out_hbm.at[base_task + t], osem)
            @pl.when(i < rows_pw - 1)
            def _():
                pltpu.async_copy(
                    ent_hbm.at[tidx_v.at[t + halves]], tails, gsem)

        def loop_body(i, carry):
            run_task(i, 0, tails0, scores0, gsem0, osem0)
            run_task(i, 1, tails1, scores1, gsem1, osem1)
            return carry
        lax.fori_loop(0, rows_pw, loop_body, 0)

        pltpu.make_async_copy(scores0, out_hbm.at[base_task], osem0).wait()
        pltpu.make_async_copy(scores1, out_hbm.at[base_task], osem1).wait()

    return k


def kernel(head_part, tail_part, relative_dist, entity_embedding,
           relation_embedding, relation_head, relation_tail):
    B, NEG = tail_part.shape
    h_idx = head_part[:, 0].astype(jnp.int32)
    r_idx = head_part[:, 1].astype(jnp.int32)
    tidx = tail_part.astype(jnp.int32).reshape(B * (NEG // 128), 128)
    k = _make_sc_kernel(B, NEG)
    out = k(h_idx, r_idx, tidx, entity_embedding, relation_embedding)
    return out.reshape(B, NEG)
